# baseline (speedup 1.0000x reference)
import functools

import jax
import jax.numpy as jnp
from jax import lax
from jax.experimental import pallas as pl
from jax.experimental.pallas import tpu as pltpu
from jax.experimental.pallas import tpu_sc as plsc

N1 = 8000
N2 = 2000
N = N1 + N2
E = 320000
D = 128
B = 1024

NC = 2
NS = 16
NW = NC * NS
L = 16

EPT = E // NW
G = 128
HITCAP = EPT + 2 * G
ACC_ROWS = 2048
DUMP = B
B_PER_TILE = B // NS


def _normalize_body(x1_ref, x2_ref, o_ref):
    i = pl.program_id(0)

    def _norm(x):
        n = jnp.sqrt(jnp.sum(x * x, axis=1, keepdims=True))
        return x / jnp.maximum(n, 1e-12)

    @pl.when(i < 8)
    def _():
        o_ref[...] = _norm(x1_ref[...])

    @pl.when(i >= 8)
    def _():
        o_ref[...] = _norm(x2_ref[...])


def _normalize(x1, x2):
    rows = 1000
    return pl.pallas_call(
        _normalize_body,
        grid=(10,),
        in_specs=[
            pl.BlockSpec((rows, D), lambda i: (jnp.minimum(i, 7), 0)),
            pl.BlockSpec((rows, D), lambda i: (jnp.maximum(i - 8, 0), 0)),
        ],
        out_specs=pl.BlockSpec((rows, D), lambda i: (i, 0)),
        out_shape=jax.ShapeDtypeStruct((N, D), jnp.float32),
    )(x1, x2)


def _combine_body(a_ref, b_ref, o_ref):
    o_ref[...] = 0.5 * (a_ref[...] + b_ref[...])


def _combine(a, b):
    return pl.pallas_call(
        _combine_body,
        out_shape=jax.ShapeDtypeStruct((B, D), jnp.float32),
    )(a, b)


def _sc_body(xnorm_hbm, edge_hbm, w_hbm, q_hbm,
             pa_hbm, pb_hbm,
             mask_v, q_v, src_v, dst_v, w_v,
             hsrc, hdst, hw, idx_g, idx2, sidx, sidx2, qidx,
             rows_v, rows2_v, xrows_v,
             acc, sem, sem2, ssem, esem):
    cid = lax.axis_index("c")
    sid = lax.axis_index("s")
    wid = cid * NS + sid

    zeros16 = jnp.zeros((L,), jnp.float32)
    izeros16 = jnp.zeros((L,), jnp.int32)
    _ns = jax.named_scope

    base = wid * EPT
    pltpu.async_copy(edge_hbm.at[pl.ds(base, EPT)], src_v, esem)
    pltpu.async_copy(edge_hbm.at[pl.ds(E + base, EPT)], dst_v, esem)
    pltpu.async_copy(w_hbm.at[pl.ds(base, EPT)], w_v, esem)

    def _zrow(r, _):
        for c in range(D // L):
            rows_v[r, pl.ds(c * L, L)] = zeros16
        return 0
    with _ns("p0_zero"):
        lax.fori_loop(0, G, _zrow, 0)
        acc_per_tile = ACC_ROWS // NS
        for k in range(acc_per_tile // G):
            pltpu.sync_copy(rows_v, acc.at[pl.ds(sid * acc_per_tile + k * G, G)])

    neg16 = jnp.full((L,), -1, jnp.int32)
    def _zmask(i, _):
        mask_v[pl.ds(i * L, L)] = neg16
        return 0
    lane = jnp.arange(L, dtype=jnp.int32)
    def _smask(i, _):
        qv = q_v[pl.ds(i * L, L)]
        plsc.store_scatter(mask_v, [qv], lane + jnp.full((L,), i * L, jnp.int32))
        return 0
    with _ns("p1_pos"):
        lax.fori_loop(0, (N + 2 * L - 1) // L, _zmask, 0, unroll=8)
        pltpu.sync_copy(q_hbm, q_v)
        lax.fori_loop(0, B // L, _smask, 0, unroll=4)

    with _ns("p2_wait_edges"):
        pltpu.make_async_copy(edge_hbm.at[pl.ds(base, EPT)], src_v, esem).wait()
        pltpu.make_async_copy(edge_hbm.at[pl.ds(E + base, EPT)], dst_v, esem).wait()
        pltpu.make_async_copy(w_hbm.at[pl.ds(base, EPT)], w_v, esem).wait()

    def _filt(i, off):
        d = dst_v[pl.ds(i * L, L)]
        p = plsc.load_gather(mask_v, [d])
        hit = p >= 0
        s = src_v[pl.ds(i * L, L)]
        wv = w_v[pl.ds(i * L, L)]
        plsc.store_compressed(hsrc.at[pl.ds(off, L)], s, mask=hit)
        plsc.store_compressed(hdst.at[pl.ds(off, L)], p, mask=hit)
        plsc.store_compressed(hw.at[pl.ds(off, L)], wv, mask=hit)
        return off + plsc.all_reduce_population_count(hit)[0]
    with _ns("p3_filter"):
        nhit = lax.fori_loop(0, EPT // L, _filt, jnp.int32(0))

    dump16 = jnp.full((L,), DUMP, jnp.int32)
    for t in range(G // L):
        hdst[pl.ds(nhit + t * L, L)] = dump16
        hsrc[pl.ds(nhit + t * L, L)] = izeros16
        hw[pl.ds(nhit + t * L, L)] = zeros16

    plsc.subcore_barrier()

    ngrp = (nhit + (G - 1)) // G

    def _fill_idx(buf, j):
        for t in range(G // L):
            buf[pl.ds(t * L, L)] = hsrc[pl.ds(j * G + t * L, L)]

    def _do_group(j, rbuf, ibuf, sxbuf, s, nrbuf, nibuf, ns):
        pltpu.make_async_copy(xnorm_hbm.at[ibuf], rbuf, s).wait()

        @pl.when(j + 1 < ngrp)
        def _():
            _fill_idx(nibuf, j + 1)
            pltpu.async_copy(xnorm_hbm.at[nibuf], nrbuf, ns)

        def _scale(r16, _2):
            wv16 = hw[pl.ds(j * G + r16 * L, L)]
            for rr in range(L):
                wvec = jnp.full((L,), wv16[rr], jnp.float32)
                r = r16 * L + rr
                for c in range(D // L):
                    rbuf[r, pl.ds(c * L, L)] = (
                        rbuf[r, pl.ds(c * L, L)] * wvec)
            return 0
        lax.fori_loop(0, G // L, _scale, 0)
        for t in range(G // L):
            sxbuf[pl.ds(t * L, L)] = hdst[pl.ds(j * G + t * L, L)]
        pltpu.sync_copy(rbuf, acc.at[sxbuf], add=True)

    @pl.when(ngrp > 0)
    def _():
        _fill_idx(idx_g, 0)
        pltpu.async_copy(xnorm_hbm.at[idx_g], rows_v, sem)

    def _grp2(k, _):
        @pl.when(2 * k < ngrp)
        def _():
            _do_group(2 * k, rows_v, idx_g, sidx, sem, rows2_v, idx2, sem2)

        @pl.when(2 * k + 1 < ngrp)
        def _():
            _do_group(2 * k + 1, rows2_v, idx2, sidx2, sem2, rows_v, idx_g, sem)
        return 0
    with _ns("p4_groups"):
        lax.fori_loop(0, (ngrp + 1) // 2, _grp2, 0)

    with _ns("p4_barrier"):
        plsc.subcore_barrier()

    with _ns("p5_out"):
        b0 = sid * B_PER_TILE
        for t in range(B_PER_TILE // L):
            qv = q_v[pl.ds(b0 + t * L, L)]
            qidx[pl.ds(t * L, L)] = plsc.load_gather(mask_v, [qv])
        out_rows = rows_v.at[pl.ds(0, B_PER_TILE)]
        pltpu.async_copy(acc.at[qidx], out_rows, sem).wait()

        @pl.when(cid == 0)
        def _():
            for t in range(B_PER_TILE // L):
                qidx[pl.ds(t * L, L)] = q_v[pl.ds(b0 + t * L, L)]
            pltpu.async_copy(xnorm_hbm.at[qidx], xrows_v, sem).wait()
            def _add(r, _2):
                for c in range(D // L):
                    rows_v[r, pl.ds(c * L, L)] = (
                        rows_v[r, pl.ds(c * L, L)] + xrows_v[r, pl.ds(c * L, L)])
                return 0
            lax.fori_loop(0, B_PER_TILE, _add, 0)
            pltpu.sync_copy(out_rows, pa_hbm.at[pl.ds(b0, B_PER_TILE)])

        @pl.when(cid == 1)
        def _():
            pltpu.sync_copy(out_rows, pb_hbm.at[pl.ds(b0, B_PER_TILE)])


def _sc_propagate(xnorm, edge_index, w, q):
    mesh = plsc.VectorSubcoreMesh(
        core_axis_name="c", subcore_axis_name="s",
        num_cores=NC, num_subcores=NS)
    f = pl.kernel(
        _sc_body,
        out_type=(
            jax.ShapeDtypeStruct((B, D), jnp.float32),
            jax.ShapeDtypeStruct((B, D), jnp.float32),
        ),
        mesh=mesh,
        compiler_params=pltpu.CompilerParams(needs_layout_passes=False),
        scratch_types=[
            pltpu.VMEM(((N + 2 * L - 1) // L * L,), jnp.int32),
            pltpu.VMEM((B,), jnp.int32),
            pltpu.VMEM((EPT,), jnp.int32),
            pltpu.VMEM((EPT,), jnp.int32),
            pltpu.VMEM((EPT,), jnp.float32),
            pltpu.VMEM((HITCAP,), jnp.int32),
            pltpu.VMEM((HITCAP,), jnp.int32),
            pltpu.VMEM((HITCAP,), jnp.float32),
            pltpu.VMEM((G,), jnp.int32),
            pltpu.VMEM((G,), jnp.int32),
            pltpu.VMEM((G,), jnp.int32),
            pltpu.VMEM((G,), jnp.int32),
            pltpu.VMEM((B_PER_TILE,), jnp.int32),
            pltpu.VMEM((G, D), jnp.float32),
            pltpu.VMEM((G, D), jnp.float32),
            pltpu.VMEM((B_PER_TILE, D), jnp.float32),
            pltpu.VMEM_SHARED((ACC_ROWS, D), jnp.float32),
            pltpu.SemaphoreType.DMA,
            pltpu.SemaphoreType.DMA,
            pltpu.SemaphoreType.DMA,
            pltpu.SemaphoreType.DMA,
        ],
    )
    return f(xnorm, edge_index, w, q)


def kernel(emb_set_node_one, emb_set_node_two, edge_index, edge_weight, inputs):
    xnorm = _normalize(emb_set_node_one, emb_set_node_two)
    q = inputs.astype(jnp.int32)
    eflat = edge_index.astype(jnp.int32).reshape(2 * E)
    pa, pb = _sc_propagate(xnorm, eflat, edge_weight, q)
    return _combine(pa, pb)


# speedup vs baseline: 1.4669x; 1.4669x over previous
import functools

import jax
import jax.numpy as jnp
from jax import lax
from jax.experimental import pallas as pl
from jax.experimental.pallas import tpu as pltpu
from jax.experimental.pallas import tpu_sc as plsc

N1 = 8000
N2 = 2000
N = N1 + N2
E = 320000
D = 128
B = 1024

NC = 2
NS = 16
NW = NC * NS
L = 16

EPT = E // NW
G = 64
HITCAP = EPT + 2 * G
ACC_ROWS = 2048
DUMP = B
B_PER_TILE = B // NS


def _normalize_body(x1_ref, x2_ref, o_ref):
    i = pl.program_id(0)

    def _norm(x):
        n = jnp.sqrt(jnp.sum(x * x, axis=1, keepdims=True))
        return x / jnp.maximum(n, 1e-12)

    @pl.when(i < 8)
    def _():
        o_ref[...] = _norm(x1_ref[...])

    @pl.when(i >= 8)
    def _():
        o_ref[...] = _norm(x2_ref[...])


def _normalize(x1, x2):
    rows = 1000
    return pl.pallas_call(
        _normalize_body,
        grid=(10,),
        in_specs=[
            pl.BlockSpec((rows, D), lambda i: (jnp.minimum(i, 7), 0)),
            pl.BlockSpec((rows, D), lambda i: (jnp.maximum(i - 8, 0), 0)),
        ],
        out_specs=pl.BlockSpec((rows, D), lambda i: (i, 0)),
        out_shape=jax.ShapeDtypeStruct((N, D), jnp.float32),
    )(x1, x2)


def _combine_body(a_ref, b_ref, o_ref):
    o_ref[...] = 0.5 * (a_ref[...] + b_ref[...])


def _combine(a, b):
    return pl.pallas_call(
        _combine_body,
        out_shape=jax.ShapeDtypeStruct((B, D), jnp.float32),
    )(a, b)


def _sc_body(xnorm_hbm, edge_hbm, w_hbm, q_hbm,
             pa_hbm, pb_hbm,
             mask_v, q_v, src_v, dst_v, w_v,
             hsrc, hdst, hw, idx_g, idx2, sidx, sidx2, qidx,
             rows_v, rows2_v, xrows_v,
             acc, sem, sem2, ssem, esem):
    cid = lax.axis_index("c")
    sid = lax.axis_index("s")
    wid = cid * NS + sid

    zeros16 = jnp.zeros((L,), jnp.float32)
    izeros16 = jnp.zeros((L,), jnp.int32)
    _ns = jax.named_scope

    base = wid * EPT
    pltpu.async_copy(edge_hbm.at[pl.ds(base, EPT)], src_v, esem)
    pltpu.async_copy(edge_hbm.at[pl.ds(E + base, EPT)], dst_v, esem)
    pltpu.async_copy(w_hbm.at[pl.ds(base, EPT)], w_v, esem)

    def _zrow(r, _):
        for c in range(D // L):
            rows_v[r, pl.ds(c * L, L)] = zeros16
        return 0
    with _ns("p0_zero"):
        lax.fori_loop(0, G, _zrow, 0)
        acc_per_tile = ACC_ROWS // NS
        for k in range(acc_per_tile // G):
            pltpu.sync_copy(rows_v, acc.at[pl.ds(sid * acc_per_tile + k * G, G)])

    neg16 = jnp.full((L,), -1, jnp.int32)
    def _zmask(i, _):
        mask_v[pl.ds(i * L, L)] = neg16
        return 0
    lane = jnp.arange(L, dtype=jnp.int32)
    def _smask(i, _):
        qv = q_v[pl.ds(i * L, L)]
        plsc.store_scatter(mask_v, [qv], lane + jnp.full((L,), i * L, jnp.int32))
        return 0
    with _ns("p1_pos"):
        lax.fori_loop(0, (N + 2 * L - 1) // L, _zmask, 0, unroll=8)
        pltpu.sync_copy(q_hbm, q_v)
        lax.fori_loop(0, B // L, _smask, 0, unroll=4)

    with _ns("p2_wait_edges"):
        pltpu.make_async_copy(edge_hbm.at[pl.ds(base, EPT)], src_v, esem).wait()
        pltpu.make_async_copy(edge_hbm.at[pl.ds(E + base, EPT)], dst_v, esem).wait()
        pltpu.make_async_copy(w_hbm.at[pl.ds(base, EPT)], w_v, esem).wait()

    def _filt(i, off):
        d = dst_v[pl.ds(i * L, L)]
        p = plsc.load_gather(mask_v, [d])
        hit = p >= 0
        s = src_v[pl.ds(i * L, L)]
        wv = w_v[pl.ds(i * L, L)]
        plsc.store_compressed(hsrc.at[pl.ds(off, L)], s, mask=hit)
        plsc.store_compressed(hdst.at[pl.ds(off, L)], p, mask=hit)
        plsc.store_compressed(hw.at[pl.ds(off, L)], wv, mask=hit)
        return off + plsc.all_reduce_population_count(hit)[0]
    with _ns("p3_filter"):
        nhit = lax.fori_loop(0, EPT // L, _filt, jnp.int32(0))

    dump16 = jnp.full((L,), DUMP, jnp.int32)
    for t in range(G // L):
        hdst[pl.ds(nhit + t * L, L)] = dump16
        hsrc[pl.ds(nhit + t * L, L)] = izeros16
        hw[pl.ds(nhit + t * L, L)] = zeros16

    plsc.subcore_barrier()

    ngrp = (nhit + (G - 1)) // G

    def _fill_idx(buf, j):
        for t in range(G // L):
            buf[pl.ds(t * L, L)] = hsrc[pl.ds(j * G + t * L, L)]

    def _do_group(j, rbuf, ibuf, sxbuf, s, nrbuf, nibuf, ns):
        pltpu.make_async_copy(xnorm_hbm.at[ibuf], rbuf, s).wait()

        @pl.when(j + 1 < ngrp)
        def _():
            _fill_idx(nibuf, j + 1)
            pltpu.async_copy(xnorm_hbm.at[nibuf], nrbuf, ns)

        def _scale(r16, _2):
            wv16 = hw[pl.ds(j * G + r16 * L, L)]
            for rr in range(L):
                wvec = jnp.full((L,), wv16[rr], jnp.float32)
                r = r16 * L + rr
                for c in range(D // L):
                    rbuf[r, pl.ds(c * L, L)] = (
                        rbuf[r, pl.ds(c * L, L)] * wvec)
            return 0
        lax.fori_loop(0, G // L, _scale, 0)
        for t in range(G // L):
            sxbuf[pl.ds(t * L, L)] = hdst[pl.ds(j * G + t * L, L)]
        pltpu.sync_copy(rbuf, acc.at[sxbuf], add=True)

    @pl.when(ngrp > 0)
    def _():
        _fill_idx(idx_g, 0)
        pltpu.async_copy(xnorm_hbm.at[idx_g], rows_v, sem)

    def _grp2(k, _):
        @pl.when(2 * k < ngrp)
        def _():
            _do_group(2 * k, rows_v, idx_g, sidx, sem, rows2_v, idx2, sem2)

        @pl.when(2 * k + 1 < ngrp)
        def _():
            _do_group(2 * k + 1, rows2_v, idx2, sidx2, sem2, rows_v, idx_g, sem)
        return 0
    with _ns("p4_groups"):
        lax.fori_loop(0, (ngrp + 1) // 2, _grp2, 0)

    with _ns("p4_barrier"):
        plsc.subcore_barrier()

    with _ns("p5_out"):
        b0 = sid * B_PER_TILE
        for t in range(B_PER_TILE // L):
            qv = q_v[pl.ds(b0 + t * L, L)]
            qidx[pl.ds(t * L, L)] = plsc.load_gather(mask_v, [qv])
        out_rows = rows_v.at[pl.ds(0, B_PER_TILE)]
        pltpu.async_copy(acc.at[qidx], out_rows, sem).wait()

        @pl.when(cid == 0)
        def _():
            for t in range(B_PER_TILE // L):
                qidx[pl.ds(t * L, L)] = q_v[pl.ds(b0 + t * L, L)]
            pltpu.async_copy(xnorm_hbm.at[qidx], xrows_v, sem).wait()
            def _add(r, _2):
                for c in range(D // L):
                    rows_v[r, pl.ds(c * L, L)] = (
                        rows_v[r, pl.ds(c * L, L)] + xrows_v[r, pl.ds(c * L, L)])
                return 0
            lax.fori_loop(0, B_PER_TILE, _add, 0)
            pltpu.sync_copy(out_rows, pa_hbm.at[pl.ds(b0, B_PER_TILE)])

        @pl.when(cid == 1)
        def _():
            pltpu.sync_copy(out_rows, pb_hbm.at[pl.ds(b0, B_PER_TILE)])


def _sc_propagate(xnorm, edge_index, w, q):
    mesh = plsc.VectorSubcoreMesh(
        core_axis_name="c", subcore_axis_name="s",
        num_cores=NC, num_subcores=NS)
    f = pl.kernel(
        _sc_body,
        out_type=(
            jax.ShapeDtypeStruct((B, D), jnp.float32),
            jax.ShapeDtypeStruct((B, D), jnp.float32),
        ),
        mesh=mesh,
        compiler_params=pltpu.CompilerParams(needs_layout_passes=False),
        scratch_types=[
            pltpu.VMEM(((N + 2 * L - 1) // L * L,), jnp.int32),
            pltpu.VMEM((B,), jnp.int32),
            pltpu.VMEM((EPT,), jnp.int32),
            pltpu.VMEM((EPT,), jnp.int32),
            pltpu.VMEM((EPT,), jnp.float32),
            pltpu.VMEM((HITCAP,), jnp.int32),
            pltpu.VMEM((HITCAP,), jnp.int32),
            pltpu.VMEM((HITCAP,), jnp.float32),
            pltpu.VMEM((G,), jnp.int32),
            pltpu.VMEM((G,), jnp.int32),
            pltpu.VMEM((G,), jnp.int32),
            pltpu.VMEM((G,), jnp.int32),
            pltpu.VMEM((B_PER_TILE,), jnp.int32),
            pltpu.VMEM((G, D), jnp.float32),
            pltpu.VMEM((G, D), jnp.float32),
            pltpu.VMEM((B_PER_TILE, D), jnp.float32),
            pltpu.VMEM_SHARED((ACC_ROWS, D), jnp.float32),
            pltpu.SemaphoreType.DMA,
            pltpu.SemaphoreType.DMA,
            pltpu.SemaphoreType.DMA,
            pltpu.SemaphoreType.DMA,
        ],
    )
    return f(xnorm, edge_index, w, q)


def kernel(emb_set_node_one, emb_set_node_two, edge_index, edge_weight, inputs):
    xnorm = _normalize(emb_set_node_one, emb_set_node_two)
    q = inputs.astype(jnp.int32)
    eflat = edge_index.astype(jnp.int32).reshape(2 * E)
    pa, pb = _sc_propagate(xnorm, eflat, edge_weight, q)
    return _combine(pa, pb)


# speedup vs baseline: 1.5815x; 1.0781x over previous
import functools

import jax
import jax.numpy as jnp
from jax import lax
from jax.experimental import pallas as pl
from jax.experimental.pallas import tpu as pltpu
from jax.experimental.pallas import tpu_sc as plsc

N1 = 8000
N2 = 2000
N = N1 + N2
E = 320000
D = 128
B = 1024

NC = 2
NS = 16
NW = NC * NS
L = 16

CH = 10112
G = 64
HITCAP = CH + 2 * G
ACC_ROWS = 2048
DUMP = B
B_PER_TILE = B // NS


def _normalize_body(x1_ref, x2_ref, o_ref):
    i = pl.program_id(0)

    def _norm(x):
        nsq = jnp.sum(x * x, axis=1, keepdims=True)
        return x * lax.rsqrt(jnp.maximum(nsq, 1e-24))

    @pl.when(i < 4)
    def _():
        o_ref[...] = _norm(x1_ref[...])

    @pl.when(i >= 4)
    def _():
        o_ref[...] = _norm(x2_ref[...])


def _normalize(x1, x2):
    rows = 2000
    return pl.pallas_call(
        _normalize_body,
        grid=(5,),
        in_specs=[
            pl.BlockSpec((rows, D), lambda i: (jnp.minimum(i, 3), 0)),
            pl.BlockSpec((rows, D), lambda i: (jnp.maximum(i - 4, 0), 0)),
        ],
        out_specs=pl.BlockSpec((rows, D), lambda i: (i, 0)),
        out_shape=jax.ShapeDtypeStruct((N, D), jnp.float32),
    )(x1, x2)


def _combine_body(a_ref, b_ref, o_ref):
    o_ref[...] = 0.5 * (a_ref[...] + b_ref[...])


def _combine(a, b):
    return pl.pallas_call(
        _combine_body,
        out_shape=jax.ShapeDtypeStruct((B, D), jnp.float32),
    )(a, b)


def _sc_body(xnorm_hbm, edge_hbm, w_hbm, q_hbm,
             pa_hbm, pb_hbm,
             mask_v, q_v, edge_v, w_v,
             hsrc, hdst, hw, idx_g, idx2, sidx, sidx2, qidx,
             rows_v, rows2_v, xrows_v,
             acc, sem, sem2, ssem, esem):
    cid = lax.axis_index("c")
    sid = lax.axis_index("s")
    wid = cid * NS + sid

    zeros16 = jnp.zeros((L,), jnp.float32)
    izeros16 = jnp.zeros((L,), jnp.int32)
    _ns = jax.named_scope

    base = jnp.minimum(wid * CH, E - CH)
    skip = (wid * CH - base) // L
    pltpu.async_copy(edge_hbm.at[:, pl.ds(base, CH)], edge_v, esem)
    pltpu.async_copy(w_hbm.at[pl.ds(base, CH)], w_v, esem)

    def _zrow(r, _):
        for c in range(D // L):
            rows_v[r, pl.ds(c * L, L)] = zeros16
        return 0
    with _ns("p0_zero"):
        lax.fori_loop(0, G, _zrow, 0)
        acc_per_tile = ACC_ROWS // NS
        for k in range(acc_per_tile // G):
            pltpu.sync_copy(rows_v, acc.at[pl.ds(sid * acc_per_tile + k * G, G)])

    neg16 = jnp.full((L,), -1, jnp.int32)
    def _zmask(i, _):
        mask_v[pl.ds(i * L, L)] = neg16
        return 0
    lane = jnp.arange(L, dtype=jnp.int32)
    def _smask(i, _):
        qv = q_v[pl.ds(i * L, L)]
        plsc.store_scatter(mask_v, [qv], lane + jnp.full((L,), i * L, jnp.int32))
        return 0
    with _ns("p1_pos"):
        lax.fori_loop(0, (N + 2 * L - 1) // L, _zmask, 0, unroll=8)
        pltpu.sync_copy(q_hbm, q_v)
        lax.fori_loop(0, B // L, _smask, 0, unroll=4)

    with _ns("p2_wait_edges"):
        pltpu.make_async_copy(edge_hbm.at[:, pl.ds(base, CH)], edge_v, esem).wait()
        pltpu.make_async_copy(w_hbm.at[pl.ds(base, CH)], w_v, esem).wait()

    def _filt(i, off):
        d = edge_v[1, pl.ds(i * L, L)]
        p = plsc.load_gather(mask_v, [d])
        hit = p >= 0
        s = edge_v[0, pl.ds(i * L, L)]
        wv = w_v[pl.ds(i * L, L)]
        plsc.store_compressed(hsrc.at[pl.ds(off, L)], s, mask=hit)
        plsc.store_compressed(hdst.at[pl.ds(off, L)], p, mask=hit)
        plsc.store_compressed(hw.at[pl.ds(off, L)], wv, mask=hit)
        return off + plsc.all_reduce_population_count(hit)[0]
    with _ns("p3_filter"):
        nhit = lax.fori_loop(skip, CH // L, _filt, jnp.int32(0))

    dump16 = jnp.full((L,), DUMP, jnp.int32)
    for t in range(G // L):
        hdst[pl.ds(nhit + t * L, L)] = dump16
        hsrc[pl.ds(nhit + t * L, L)] = izeros16
        hw[pl.ds(nhit + t * L, L)] = zeros16

    plsc.subcore_barrier()

    ngrp = (nhit + (G - 1)) // G

    def _fill_idx(buf, j):
        for t in range(G // L):
            buf[pl.ds(t * L, L)] = hsrc[pl.ds(j * G + t * L, L)]

    def _do_group(j, rbuf, ibuf, sxbuf, s, nrbuf, nibuf, ns):
        pltpu.make_async_copy(xnorm_hbm.at[ibuf], rbuf, s).wait()

        @pl.when(j + 1 < ngrp)
        def _():
            _fill_idx(nibuf, j + 1)
            pltpu.async_copy(xnorm_hbm.at[nibuf], nrbuf, ns)

        def _scale(r16, _2):
            wv16 = hw[pl.ds(j * G + r16 * L, L)]
            for rr in range(L):
                wvec = jnp.full((L,), wv16[rr], jnp.float32)
                r = r16 * L + rr
                for c in range(D // L):
                    rbuf[r, pl.ds(c * L, L)] = (
                        rbuf[r, pl.ds(c * L, L)] * wvec)
            return 0
        lax.fori_loop(0, G // L, _scale, 0)
        for t in range(G // L):
            sxbuf[pl.ds(t * L, L)] = hdst[pl.ds(j * G + t * L, L)]
        pltpu.sync_copy(rbuf, acc.at[sxbuf], add=True)

    @pl.when(ngrp > 0)
    def _():
        _fill_idx(idx_g, 0)
        pltpu.async_copy(xnorm_hbm.at[idx_g], rows_v, sem)

    def _grp2(k, _):
        @pl.when(2 * k < ngrp)
        def _():
            _do_group(2 * k, rows_v, idx_g, sidx, sem, rows2_v, idx2, sem2)

        @pl.when(2 * k + 1 < ngrp)
        def _():
            _do_group(2 * k + 1, rows2_v, idx2, sidx2, sem2, rows_v, idx_g, sem)
        return 0
    with _ns("p4_groups"):
        lax.fori_loop(0, (ngrp + 1) // 2, _grp2, 0)

    with _ns("p4_barrier"):
        plsc.subcore_barrier()

    with _ns("p5_out"):
        b0 = sid * B_PER_TILE
        for t in range(B_PER_TILE // L):
            qv = q_v[pl.ds(b0 + t * L, L)]
            qidx[pl.ds(t * L, L)] = plsc.load_gather(mask_v, [qv])
        out_rows = rows_v.at[pl.ds(0, B_PER_TILE)]
        pltpu.async_copy(acc.at[qidx], out_rows, sem).wait()

        @pl.when(cid == 0)
        def _():
            for t in range(B_PER_TILE // L):
                qidx[pl.ds(t * L, L)] = q_v[pl.ds(b0 + t * L, L)]
            pltpu.async_copy(xnorm_hbm.at[qidx], xrows_v, sem).wait()
            def _add(r, _2):
                for c in range(D // L):
                    rows_v[r, pl.ds(c * L, L)] = (
                        rows_v[r, pl.ds(c * L, L)] + xrows_v[r, pl.ds(c * L, L)])
                return 0
            lax.fori_loop(0, B_PER_TILE, _add, 0)
            pltpu.sync_copy(out_rows, pa_hbm.at[pl.ds(b0, B_PER_TILE)])

        @pl.when(cid == 1)
        def _():
            pltpu.sync_copy(out_rows, pb_hbm.at[pl.ds(b0, B_PER_TILE)])


def _sc_propagate(xnorm, edge_index, w, q):
    mesh = plsc.VectorSubcoreMesh(
        core_axis_name="c", subcore_axis_name="s",
        num_cores=NC, num_subcores=NS)
    f = pl.kernel(
        _sc_body,
        out_type=(
            jax.ShapeDtypeStruct((B, D), jnp.float32),
            jax.ShapeDtypeStruct((B, D), jnp.float32),
        ),
        mesh=mesh,
        compiler_params=pltpu.CompilerParams(needs_layout_passes=False),
        scratch_types=[
            pltpu.VMEM(((N + 2 * L - 1) // L * L,), jnp.int32),
            pltpu.VMEM((B,), jnp.int32),
            pltpu.VMEM((2, CH), jnp.int32),
            pltpu.VMEM((CH,), jnp.float32),
            pltpu.VMEM((HITCAP,), jnp.int32),
            pltpu.VMEM((HITCAP,), jnp.int32),
            pltpu.VMEM((HITCAP,), jnp.float32),
            pltpu.VMEM((G,), jnp.int32),
            pltpu.VMEM((G,), jnp.int32),
            pltpu.VMEM((G,), jnp.int32),
            pltpu.VMEM((G,), jnp.int32),
            pltpu.VMEM((B_PER_TILE,), jnp.int32),
            pltpu.VMEM((G, D), jnp.float32),
            pltpu.VMEM((G, D), jnp.float32),
            pltpu.VMEM((B_PER_TILE, D), jnp.float32),
            pltpu.VMEM_SHARED((ACC_ROWS, D), jnp.float32),
            pltpu.SemaphoreType.DMA,
            pltpu.SemaphoreType.DMA,
            pltpu.SemaphoreType.DMA,
            pltpu.SemaphoreType.DMA,
        ],
    )
    return f(xnorm, edge_index, w, q)


def kernel(emb_set_node_one, emb_set_node_two, edge_index, edge_weight, inputs):
    xnorm = _normalize(emb_set_node_one, emb_set_node_two)
    q = inputs.astype(jnp.int32)
    pa, pb = _sc_propagate(xnorm, edge_index.astype(jnp.int32), edge_weight, q)
    return _combine(pa, pb)


# speedup vs baseline: 1.5827x; 1.0008x over previous
import functools

import jax
import jax.numpy as jnp
from jax import lax
from jax.experimental import pallas as pl
from jax.experimental.pallas import tpu as pltpu
from jax.experimental.pallas import tpu_sc as plsc

N1 = 8000
N2 = 2000
N = N1 + N2
E = 320000
D = 128
B = 1024

NC = 2
NS = 16
NW = NC * NS
L = 16

CH = 10112
G = 64
HITCAP = CH + 2 * G
ACC_ROWS = 2048
DUMP = B
B_PER_TILE = B // NS


def _normalize_body(x1_ref, x2_ref, o_ref):
    i = pl.program_id(0)

    def _norm(x):
        nsq = jnp.sum(x * x, axis=1, keepdims=True)
        return x * lax.rsqrt(jnp.maximum(nsq, 1e-24))

    @pl.when(i < 4)
    def _():
        o_ref[...] = _norm(x1_ref[...])

    @pl.when(i >= 4)
    def _():
        o_ref[...] = _norm(x2_ref[...])


def _normalize(x1, x2):
    rows = 2000
    return pl.pallas_call(
        _normalize_body,
        grid=(5,),
        in_specs=[
            pl.BlockSpec((rows, D), lambda i: (jnp.minimum(i, 3), 0)),
            pl.BlockSpec((rows, D), lambda i: (jnp.maximum(i - 4, 0), 0)),
        ],
        out_specs=pl.BlockSpec((rows, D), lambda i: (i, 0)),
        out_shape=jax.ShapeDtypeStruct((N, D), jnp.float32),
    )(x1, x2)


def _combine_body(a_ref, b_ref, o_ref):
    o_ref[...] = 0.5 * (a_ref[...] + b_ref[...])


def _combine(a, b):
    return pl.pallas_call(
        _combine_body,
        out_shape=jax.ShapeDtypeStruct((B, D), jnp.float32),
    )(a, b)


def _sc_body(xnorm_hbm, edge_hbm, w_hbm, q_hbm,
             pa_hbm, pb_hbm,
             mask_v, q_v, edge_v, w_v,
             hsrc, hdst, hw, idx_g, idx2, sidx, sidx2, qidx,
             rows_v, rows2_v, xrows_v,
             acc, sem, sem2, ssem, esem):
    cid = lax.axis_index("c")
    sid = lax.axis_index("s")
    wid = cid * NS + sid

    zeros16 = jnp.zeros((L,), jnp.float32)
    izeros16 = jnp.zeros((L,), jnp.int32)
    _ns = jax.named_scope

    base = jnp.minimum(wid * CH, E - CH)
    skip = (wid * CH - base) // L
    pltpu.async_copy(edge_hbm.at[:, pl.ds(base, CH)], edge_v, esem)
    pltpu.async_copy(w_hbm.at[pl.ds(base, CH)], w_v, esem)

    def _zrow(r, _):
        for c in range(D // L):
            rows_v[r, pl.ds(c * L, L)] = zeros16
        return 0
    with _ns("p0_zero"):
        lax.fori_loop(0, G, _zrow, 0)
        acc_per_tile = ACC_ROWS // NS
        for k in range(acc_per_tile // G):
            pltpu.sync_copy(rows_v, acc.at[pl.ds(sid * acc_per_tile + k * G, G)])

    neg16 = jnp.full((L,), -1, jnp.int32)
    def _zmask(i, _):
        mask_v[pl.ds(i * L, L)] = neg16
        return 0
    lane = jnp.arange(L, dtype=jnp.int32)
    def _smask(i, _):
        qv = q_v[pl.ds(i * L, L)]
        plsc.store_scatter(mask_v, [qv], lane + jnp.full((L,), i * L, jnp.int32))
        return 0
    with _ns("p1_pos"):
        lax.fori_loop(0, (N + 2 * L - 1) // L, _zmask, 0, unroll=8)
        pltpu.sync_copy(q_hbm, q_v)
        lax.fori_loop(0, B // L, _smask, 0, unroll=4)

    with _ns("p2_wait_edges"):
        pltpu.make_async_copy(edge_hbm.at[:, pl.ds(base, CH)], edge_v, esem).wait()
        pltpu.make_async_copy(w_hbm.at[pl.ds(base, CH)], w_v, esem).wait()

    def _filt(i, off):
        d = edge_v[1, pl.ds(i * L, L)]
        p = plsc.load_gather(mask_v, [d])
        hit = p >= 0
        s = edge_v[0, pl.ds(i * L, L)]
        wv = w_v[pl.ds(i * L, L)]
        plsc.store_compressed(hsrc.at[pl.ds(off, L)], s, mask=hit)
        plsc.store_compressed(hdst.at[pl.ds(off, L)], p, mask=hit)
        plsc.store_compressed(hw.at[pl.ds(off, L)], wv, mask=hit)
        return off + plsc.all_reduce_population_count(hit)[0]
    with _ns("p3_filter"):
        nhit = lax.fori_loop(skip, CH // L, _filt, jnp.int32(0))

    dump16 = jnp.full((L,), DUMP, jnp.int32)
    for t in range(G // L):
        hdst[pl.ds(nhit + t * L, L)] = dump16
        hsrc[pl.ds(nhit + t * L, L)] = izeros16
        hw[pl.ds(nhit + t * L, L)] = zeros16

    plsc.subcore_barrier()

    ngrp = (nhit + (G - 1)) // G

    def _fill_idx(buf, j):
        for t in range(G // L):
            buf[pl.ds(t * L, L)] = hsrc[pl.ds(j * G + t * L, L)]

    def _do_group(j, rbuf, ibuf, sxbuf, s, nrbuf, nibuf, ns):
        with _ns("p4a_wait"):
            pltpu.make_async_copy(xnorm_hbm.at[ibuf], rbuf, s).wait()

        with _ns("p4b_prefetch"):
            @pl.when(j + 1 < ngrp)
            def _():
                _fill_idx(nibuf, j + 1)
                pltpu.async_copy(xnorm_hbm.at[nibuf], nrbuf, ns)

        def _scale(r16, _2):
            wv16 = hw[pl.ds(j * G + r16 * L, L)]
            for rr in range(L):
                wvec = jnp.full((L,), wv16[rr], jnp.float32)
                r = r16 * L + rr
                for c in range(D // L):
                    rbuf[r, pl.ds(c * L, L)] = (
                        rbuf[r, pl.ds(c * L, L)] * wvec)
            return 0
        with _ns("p4c_scale"):
            lax.fori_loop(0, G // L, _scale, 0)
        with _ns("p4d_scatter"):
            for t in range(G // L):
                sxbuf[pl.ds(t * L, L)] = hdst[pl.ds(j * G + t * L, L)]
            pltpu.sync_copy(rbuf, acc.at[sxbuf], add=True)

    @pl.when(ngrp > 0)
    def _():
        _fill_idx(idx_g, 0)
        pltpu.async_copy(xnorm_hbm.at[idx_g], rows_v, sem)

    def _grp2(k, _):
        @pl.when(2 * k < ngrp)
        def _():
            _do_group(2 * k, rows_v, idx_g, sidx, sem, rows2_v, idx2, sem2)

        @pl.when(2 * k + 1 < ngrp)
        def _():
            _do_group(2 * k + 1, rows2_v, idx2, sidx2, sem2, rows_v, idx_g, sem)
        return 0
    with _ns("p4_groups"):
        lax.fori_loop(0, (ngrp + 1) // 2, _grp2, 0)

    with _ns("p4_barrier"):
        plsc.subcore_barrier()

    with _ns("p5_out"):
        b0 = sid * B_PER_TILE
        for t in range(B_PER_TILE // L):
            qv = q_v[pl.ds(b0 + t * L, L)]
            qidx[pl.ds(t * L, L)] = plsc.load_gather(mask_v, [qv])
        out_rows = rows_v.at[pl.ds(0, B_PER_TILE)]
        pltpu.async_copy(acc.at[qidx], out_rows, sem).wait()

        @pl.when(cid == 0)
        def _():
            for t in range(B_PER_TILE // L):
                qidx[pl.ds(t * L, L)] = q_v[pl.ds(b0 + t * L, L)]
            pltpu.async_copy(xnorm_hbm.at[qidx], xrows_v, sem).wait()
            def _add(r, _2):
                for c in range(D // L):
                    rows_v[r, pl.ds(c * L, L)] = (
                        rows_v[r, pl.ds(c * L, L)] + xrows_v[r, pl.ds(c * L, L)])
                return 0
            lax.fori_loop(0, B_PER_TILE, _add, 0)
            pltpu.sync_copy(out_rows, pa_hbm.at[pl.ds(b0, B_PER_TILE)])

        @pl.when(cid == 1)
        def _():
            pltpu.sync_copy(out_rows, pb_hbm.at[pl.ds(b0, B_PER_TILE)])


def _sc_propagate(xnorm, edge_index, w, q):
    mesh = plsc.VectorSubcoreMesh(
        core_axis_name="c", subcore_axis_name="s",
        num_cores=NC, num_subcores=NS)
    f = pl.kernel(
        _sc_body,
        out_type=(
            jax.ShapeDtypeStruct((B, D), jnp.float32),
            jax.ShapeDtypeStruct((B, D), jnp.float32),
        ),
        mesh=mesh,
        compiler_params=pltpu.CompilerParams(needs_layout_passes=False),
        scratch_types=[
            pltpu.VMEM(((N + 2 * L - 1) // L * L,), jnp.int32),
            pltpu.VMEM((B,), jnp.int32),
            pltpu.VMEM((2, CH), jnp.int32),
            pltpu.VMEM((CH,), jnp.float32),
            pltpu.VMEM((HITCAP,), jnp.int32),
            pltpu.VMEM((HITCAP,), jnp.int32),
            pltpu.VMEM((HITCAP,), jnp.float32),
            pltpu.VMEM((G,), jnp.int32),
            pltpu.VMEM((G,), jnp.int32),
            pltpu.VMEM((G,), jnp.int32),
            pltpu.VMEM((G,), jnp.int32),
            pltpu.VMEM((B_PER_TILE,), jnp.int32),
            pltpu.VMEM((G, D), jnp.float32),
            pltpu.VMEM((G, D), jnp.float32),
            pltpu.VMEM((B_PER_TILE, D), jnp.float32),
            pltpu.VMEM_SHARED((ACC_ROWS, D), jnp.float32),
            pltpu.SemaphoreType.DMA,
            pltpu.SemaphoreType.DMA,
            pltpu.SemaphoreType.DMA,
            pltpu.SemaphoreType.DMA,
        ],
    )
    return f(xnorm, edge_index, w, q)


def kernel(emb_set_node_one, emb_set_node_two, edge_index, edge_weight, inputs):
    xnorm = _normalize(emb_set_node_one, emb_set_node_two)
    q = inputs.astype(jnp.int32)
    pa, pb = _sc_propagate(xnorm, edge_index.astype(jnp.int32), edge_weight, q)
    return _combine(pa, pb)


# speedup vs baseline: 1.6310x; 1.0305x over previous
import functools

import jax
import jax.numpy as jnp
from jax import lax
from jax.experimental import pallas as pl
from jax.experimental.pallas import tpu as pltpu
from jax.experimental.pallas import tpu_sc as plsc

N1 = 8000
N2 = 2000
N = N1 + N2
E = 320000
D = 128
B = 1024

NC = 2
NS = 16
NW = NC * NS
L = 16

CH = 10112
G = 64
HITCAP = CH + 2 * G
ACC_ROWS = 2048
DUMP = B
B_PER_TILE = B // NS


def _normalize_body(x1_ref, x2_ref, o_ref):
    i = pl.program_id(0)

    def _norm(x):
        nsq = jnp.sum(x * x, axis=1, keepdims=True)
        return x * lax.rsqrt(jnp.maximum(nsq, 1e-24))

    @pl.when(i < 4)
    def _():
        o_ref[...] = _norm(x1_ref[...])

    @pl.when(i >= 4)
    def _():
        o_ref[...] = _norm(x2_ref[...])


def _normalize(x1, x2):
    rows = 2000
    return pl.pallas_call(
        _normalize_body,
        grid=(5,),
        in_specs=[
            pl.BlockSpec((rows, D), lambda i: (jnp.minimum(i, 3), 0)),
            pl.BlockSpec((rows, D), lambda i: (jnp.maximum(i - 4, 0), 0)),
        ],
        out_specs=pl.BlockSpec((rows, D), lambda i: (i, 0)),
        out_shape=jax.ShapeDtypeStruct((N, D), jnp.float32),
    )(x1, x2)


def _combine_body(a_ref, b_ref, o_ref):
    o_ref[...] = 0.5 * (a_ref[...] + b_ref[...])


def _combine(a, b):
    return pl.pallas_call(
        _combine_body,
        out_shape=jax.ShapeDtypeStruct((B, D), jnp.float32),
    )(a, b)


def _sc_body(xnorm_hbm, edge_hbm, w_hbm, q_hbm,
             pa_hbm, pb_hbm,
             mask_v, q_v, edge_v, w_v,
             hsrc, hdst, hw, idx_g, idx2, idx3, sidx, sidx2, sidx3, qidx,
             rows_v, rows2_v, rows3_v, xrows_v,
             acc, sem, sem2, sem3, esem):
    cid = lax.axis_index("c")
    sid = lax.axis_index("s")
    wid = cid * NS + sid

    zeros16 = jnp.zeros((L,), jnp.float32)
    izeros16 = jnp.zeros((L,), jnp.int32)
    _ns = jax.named_scope

    base = jnp.minimum(wid * CH, E - CH)
    skip = (wid * CH - base) // L
    pltpu.async_copy(edge_hbm.at[:, pl.ds(base, CH)], edge_v, esem)
    pltpu.async_copy(w_hbm.at[pl.ds(base, CH)], w_v, esem)

    def _zrow(r, _):
        for c in range(D // L):
            rows_v[r, pl.ds(c * L, L)] = zeros16
        return 0
    with _ns("p0_zero"):
        lax.fori_loop(0, G, _zrow, 0)
        acc_per_tile = ACC_ROWS // NS
        for k in range(acc_per_tile // G):
            pltpu.sync_copy(rows_v, acc.at[pl.ds(sid * acc_per_tile + k * G, G)])

    neg16 = jnp.full((L,), -1, jnp.int32)
    def _zmask(i, _):
        mask_v[pl.ds(i * L, L)] = neg16
        return 0
    lane = jnp.arange(L, dtype=jnp.int32)
    def _smask(i, _):
        qv = q_v[pl.ds(i * L, L)]
        plsc.store_scatter(mask_v, [qv], lane + jnp.full((L,), i * L, jnp.int32))
        return 0
    with _ns("p1_pos"):
        lax.fori_loop(0, (N + 2 * L - 1) // L, _zmask, 0, unroll=8)
        pltpu.sync_copy(q_hbm, q_v)
        lax.fori_loop(0, B // L, _smask, 0, unroll=4)

    with _ns("p2_wait_edges"):
        pltpu.make_async_copy(edge_hbm.at[:, pl.ds(base, CH)], edge_v, esem).wait()
        pltpu.make_async_copy(w_hbm.at[pl.ds(base, CH)], w_v, esem).wait()

    def _filt(i, off):
        d = edge_v[1, pl.ds(i * L, L)]
        p = plsc.load_gather(mask_v, [d])
        hit = p >= 0
        s = edge_v[0, pl.ds(i * L, L)]
        wv = w_v[pl.ds(i * L, L)]
        plsc.store_compressed(hsrc.at[pl.ds(off, L)], s, mask=hit)
        plsc.store_compressed(hdst.at[pl.ds(off, L)], p, mask=hit)
        plsc.store_compressed(hw.at[pl.ds(off, L)], wv, mask=hit)
        return off + plsc.all_reduce_population_count(hit)[0]
    with _ns("p3_filter"):
        nhit = lax.fori_loop(skip, CH // L, _filt, jnp.int32(0))

    dump16 = jnp.full((L,), DUMP, jnp.int32)
    for t in range(G // L):
        hdst[pl.ds(nhit + t * L, L)] = dump16
        hsrc[pl.ds(nhit + t * L, L)] = izeros16
        hw[pl.ds(nhit + t * L, L)] = zeros16

    plsc.subcore_barrier()

    ngrp = (nhit + (G - 1)) // G

    def _fill_idx(buf, j):
        for t in range(G // L):
            buf[pl.ds(t * L, L)] = hsrc[pl.ds(j * G + t * L, L)]

    rbufs = (rows_v, rows2_v, rows3_v)
    ibufs = (idx_g, idx2, idx3)
    sxbufs = (sidx, sidx2, sidx3)
    sems = (sem, sem2, sem3)
    NBUF = 3

    def _do_group(j, slot):
        rbuf, ibuf, sxbuf, s = rbufs[slot], ibufs[slot], sxbufs[slot], sems[slot]
        with _ns("p4a_wait"):
            pltpu.make_async_copy(xnorm_hbm.at[ibuf], rbuf, s).wait()

        with _ns("p4b_prefetch"):
            @pl.when(j + NBUF - 1 < ngrp)
            def _():
                nslot = (slot + NBUF - 1) % NBUF
                _fill_idx(ibufs[nslot], j + NBUF - 1)
                pltpu.async_copy(
                    xnorm_hbm.at[ibufs[nslot]], rbufs[nslot], sems[nslot])

        def _scale(r16, _2):
            wv16 = hw[pl.ds(j * G + r16 * L, L)]
            for rr in range(L):
                wvec = jnp.full((L,), wv16[rr], jnp.float32)
                r = r16 * L + rr
                for c in range(D // L):
                    rbuf[r, pl.ds(c * L, L)] = (
                        rbuf[r, pl.ds(c * L, L)] * wvec)
            return 0
        with _ns("p4c_scale"):
            lax.fori_loop(0, G // L, _scale, 0)
        with _ns("p4d_scatter"):
            for t in range(G // L):
                sxbuf[pl.ds(t * L, L)] = hdst[pl.ds(j * G + t * L, L)]
            pltpu.sync_copy(rbuf, acc.at[sxbuf], add=True)

    for jj in range(NBUF - 1):
        @pl.when(jj < ngrp)
        def _(jj=jj):
            _fill_idx(ibufs[jj], jj)
            pltpu.async_copy(xnorm_hbm.at[ibufs[jj]], rbufs[jj], sems[jj])

    def _grpN(k, _):
        for ph in range(NBUF):
            @pl.when(NBUF * k + ph < ngrp)
            def _(ph=ph):
                _do_group(NBUF * k + ph, ph)
        return 0
    with _ns("p4_groups"):
        lax.fori_loop(0, (ngrp + NBUF - 1) // NBUF, _grpN, 0)

    with _ns("p4_barrier"):
        plsc.subcore_barrier()

    with _ns("p5_out"):
        b0 = sid * B_PER_TILE
        for t in range(B_PER_TILE // L):
            qv = q_v[pl.ds(b0 + t * L, L)]
            qidx[pl.ds(t * L, L)] = plsc.load_gather(mask_v, [qv])
        out_rows = rows_v.at[pl.ds(0, B_PER_TILE)]
        pltpu.async_copy(acc.at[qidx], out_rows, sem).wait()

        @pl.when(cid == 0)
        def _():
            for t in range(B_PER_TILE // L):
                qidx[pl.ds(t * L, L)] = q_v[pl.ds(b0 + t * L, L)]
            pltpu.async_copy(xnorm_hbm.at[qidx], xrows_v, sem).wait()
            def _add(r, _2):
                for c in range(D // L):
                    rows_v[r, pl.ds(c * L, L)] = (
                        rows_v[r, pl.ds(c * L, L)] + xrows_v[r, pl.ds(c * L, L)])
                return 0
            lax.fori_loop(0, B_PER_TILE, _add, 0)
            pltpu.sync_copy(out_rows, pa_hbm.at[pl.ds(b0, B_PER_TILE)])

        @pl.when(cid == 1)
        def _():
            pltpu.sync_copy(out_rows, pb_hbm.at[pl.ds(b0, B_PER_TILE)])


def _sc_propagate(xnorm, edge_index, w, q):
    mesh = plsc.VectorSubcoreMesh(
        core_axis_name="c", subcore_axis_name="s",
        num_cores=NC, num_subcores=NS)
    f = pl.kernel(
        _sc_body,
        out_type=(
            jax.ShapeDtypeStruct((B, D), jnp.float32),
            jax.ShapeDtypeStruct((B, D), jnp.float32),
        ),
        mesh=mesh,
        compiler_params=pltpu.CompilerParams(needs_layout_passes=False),
        scratch_types=[
            pltpu.VMEM(((N + 2 * L - 1) // L * L,), jnp.int32),
            pltpu.VMEM((B,), jnp.int32),
            pltpu.VMEM((2, CH), jnp.int32),
            pltpu.VMEM((CH,), jnp.float32),
            pltpu.VMEM((HITCAP,), jnp.int32),
            pltpu.VMEM((HITCAP,), jnp.int32),
            pltpu.VMEM((HITCAP,), jnp.float32),
            pltpu.VMEM((G,), jnp.int32),
            pltpu.VMEM((G,), jnp.int32),
            pltpu.VMEM((G,), jnp.int32),
            pltpu.VMEM((G,), jnp.int32),
            pltpu.VMEM((G,), jnp.int32),
            pltpu.VMEM((G,), jnp.int32),
            pltpu.VMEM((B_PER_TILE,), jnp.int32),
            pltpu.VMEM((G, D), jnp.float32),
            pltpu.VMEM((G, D), jnp.float32),
            pltpu.VMEM((G, D), jnp.float32),
            pltpu.VMEM((B_PER_TILE, D), jnp.float32),
            pltpu.VMEM_SHARED((ACC_ROWS, D), jnp.float32),
            pltpu.SemaphoreType.DMA,
            pltpu.SemaphoreType.DMA,
            pltpu.SemaphoreType.DMA,
            pltpu.SemaphoreType.DMA,
        ],
    )
    return f(xnorm, edge_index, w, q)


def kernel(emb_set_node_one, emb_set_node_two, edge_index, edge_weight, inputs):
    xnorm = _normalize(emb_set_node_one, emb_set_node_two)
    q = inputs.astype(jnp.int32)
    pa, pb = _sc_propagate(xnorm, edge_index.astype(jnp.int32), edge_weight, q)
    return _combine(pa, pb)
